# Initial kernel scaffold; baseline (speedup 1.0000x reference)
import functools

import jax
import jax.numpy as jnp
from jax.experimental import pallas as pl

N = 50000
E = 400000
H = 3


def _mm_body(x_ref, w_ref, o_ref):
    o_ref[...] = jnp.dot(x_ref[...], w_ref[...], preferred_element_type=jnp.float32)


def _matmul(x, w):
    m, k = x.shape
    _, n = w.shape
    bm = 2000
    return pl.pallas_call(
        _mm_body,
        grid=(m // bm,),
        in_specs=[
            pl.BlockSpec((bm, k), lambda i: (i, 0)),
            pl.BlockSpec((k, n), lambda i: (0, 0)),
        ],
        out_specs=pl.BlockSpec((bm, n), lambda i: (i, 0)),
        out_shape=jax.ShapeDtypeStruct((m, n), jnp.float32),
    )(x, w)


def _gat(x, src, dst, W, al, ar, b, out_dim, relu):
    feat = _matmul(x, W).reshape(N, H, out_dim)
    el = jnp.sum(feat * al[None, :, :], axis=-1)
    er = jnp.sum(feat * ar[None, :, :], axis=-1)
    e = jax.nn.leaky_relu(el[src] + er[dst], negative_slope=0.2)
    ee = jnp.exp(e)
    den = jax.ops.segment_sum(ee, dst, num_segments=N)
    num = jax.ops.segment_sum(feat[src] * ee[:, :, None], dst, num_segments=N)
    out = num / (den[:, :, None] + 1e-9)
    out = out + b.reshape(1, H, out_dim)
    if relu:
        out = jax.nn.relu(out)
    return jnp.mean(out, axis=1)


def kernel(x, edge_index_rel0, edge_index_rel1,
           W0_rel0, al0_rel0, ar0_rel0, b0_rel0,
           W0_rel1, al0_rel1, ar0_rel1, b0_rel1,
           W1_rel0, al1_rel0, ar1_rel0, b1_rel0,
           W1_rel1, al1_rel1, ar1_rel1, b1_rel1):
    s0 = edge_index_rel0[0].astype(jnp.int32)
    d0 = edge_index_rel0[1].astype(jnp.int32)
    s1 = edge_index_rel1[0].astype(jnp.int32)
    d1 = edge_index_rel1[1].astype(jnp.int32)
    h0 = _gat(x, s0, d0, W0_rel0, al0_rel0, ar0_rel0, b0_rel0, 128, True)
    h1 = _gat(x, s1, d1, W0_rel1, al0_rel1, ar0_rel1, b0_rel1, 128, True)
    h = (h0 + h1) * 0.5
    o0 = _gat(h, s0, d0, W1_rel0, al1_rel0, ar1_rel0, b1_rel0, 64, False)
    o1 = _gat(h, s1, d1, W1_rel1, al1_rel1, ar1_rel1, b1_rel1, 64, False)
    return (o0 + o1) * 0.5


# speedup vs baseline: 1.0395x; 1.0395x over previous
import functools

import jax
import jax.numpy as jnp
from jax._src import config as _jax_config
from jax.experimental import pallas as pl

N = 50000
E = 400000
H = 3


def _mm_body(x_ref, w_ref, o_ref):
    o_ref[...] = jnp.dot(x_ref[...], w_ref[...], preferred_element_type=jnp.float32)


def _matmul(x, w):
    m, k = x.shape
    _, n = w.shape
    bm = 2000
    return pl.pallas_call(
        _mm_body,
        grid=(m // bm,),
        in_specs=[
            pl.BlockSpec((bm, k), lambda i: (i, 0)),
            pl.BlockSpec((k, n), lambda i: (0, 0)),
        ],
        out_specs=pl.BlockSpec((bm, n), lambda i: (i, 0)),
        out_shape=jax.ShapeDtypeStruct((m, n), jnp.float32),
    )(x, w)


def _gat(x, src, dst, W, al, ar, b, out_dim, relu):
    feat = _matmul(x, W).reshape(N, H, out_dim)
    el = jnp.sum(feat * al[None, :, :], axis=-1)
    er = jnp.sum(feat * ar[None, :, :], axis=-1)
    e = jax.nn.leaky_relu(el[src] + er[dst], negative_slope=0.2)
    ee = jnp.exp(e)
    den = jax.ops.segment_sum(ee, dst, num_segments=N)
    num = jax.ops.segment_sum(feat[src] * ee[:, :, None], dst, num_segments=N)
    out = num / (den[:, :, None] + 1e-9)
    out = out + b.reshape(1, H, out_dim)
    if relu:
        out = jax.nn.relu(out)
    return jnp.mean(out, axis=1)


def kernel(x, edge_index_rel0, edge_index_rel1,
           W0_rel0, al0_rel0, ar0_rel0, b0_rel0,
           W0_rel1, al0_rel1, ar0_rel1, b0_rel1,
           W1_rel0, al1_rel0, ar1_rel0, b1_rel0,
           W1_rel1, al1_rel1, ar1_rel1, b1_rel1):
    with _jax_config.enable_x64(False):
        s0 = edge_index_rel0[0].astype(jnp.int32)
        d0 = edge_index_rel0[1].astype(jnp.int32)
        s1 = edge_index_rel1[0].astype(jnp.int32)
        d1 = edge_index_rel1[1].astype(jnp.int32)
        h0 = _gat(x, s0, d0, W0_rel0, al0_rel0, ar0_rel0, b0_rel0, 128, True)
        h1 = _gat(x, s1, d1, W0_rel1, al0_rel1, ar0_rel1, b0_rel1, 128, True)
        h = (h0 + h1) * 0.5
        o0 = _gat(h, s0, d0, W1_rel0, al1_rel0, ar1_rel0, b1_rel0, 64, False)
        o1 = _gat(h, s1, d1, W1_rel1, al1_rel1, ar1_rel1, b1_rel1, 64, False)
        return (o0 + o1) * 0.5
